# baseline (speedup 1.0000x reference)
import functools

import jax
import jax.numpy as jnp
from jax import lax
from jax.experimental import pallas as pl
from jax.experimental.pallas import tpu as pltpu
from jax.experimental.pallas import tpu_sc as plsc

ROWS = 128
COLS = 32768
NC = 2
NS = 16
L = 16
NW = NC * NS
STEPS = COLS // L

SC_ROWS = 32
SC_BASE = ROWS - SC_ROWS
RPW = SC_ROWS // NW

TC_ROWS = ROWS - SC_ROWS
BR = 48
BC = 128

_mesh = plsc.VectorSubcoreMesh(core_axis_name="c", subcore_axis_name="s")


@functools.partial(
    pl.kernel,
    mesh=_mesh,
    out_type=jax.ShapeDtypeStruct((NW, L), jnp.int32),
    scratch_types=[
        pltpu.VMEM((2, COLS), jnp.float32),
        pltpu.VMEM((L,), jnp.int32),
        pltpu.SemaphoreType.DMA,
        pltpu.SemaphoreType.DMA,
    ],
)
def _argmax_sc(x_hbm, out_hbm, buf, res, sem0, sem1):
    wid = lax.axis_index("s") * NC + lax.axis_index("c")
    base = SC_BASE + wid * RPW
    sems = (sem0, sem1)

    copies = [pltpu.async_copy(x_hbm.at[base], buf.at[0], sems[0])]
    iota = lax.iota(jnp.int32, L)
    ansvec = jnp.zeros((L,), jnp.int32)

    for r in range(RPW):
        if r + 1 < RPW:
            copies.append(
                pltpu.async_copy(
                    x_hbm.at[base + (r + 1)], buf.at[(r + 1) % 2], sems[(r + 1) % 2]
                )
            )
        copies[r].wait()
        row = buf.at[r % 2]

        def body(j, carry):
            vmax, vstep = carry
            v = row[pl.ds(j * L, L)]
            m = v > vmax
            vmax = jnp.where(m, v, vmax)
            vstep = jnp.where(m, j, vstep)
            return vmax, vstep

        init = (
            jnp.full((L,), -jnp.inf, jnp.float32),
            jnp.zeros((L,), jnp.int32),
        )
        vmax, vstep = lax.fori_loop(0, STEPS, body, init, unroll=8)
        vidx = vstep * L + iota

        gmax = vmax
        for shift in (1, 2, 4, 8):
            perm = iota ^ shift
            gmax = jnp.maximum(gmax, gmax.at[perm].get(mode="promise_in_bounds"))
        cand = jnp.where(vmax == gmax, vidx, COLS)
        for shift in (1, 2, 4, 8):
            perm = iota ^ shift
            cand = jnp.minimum(cand, cand.at[perm].get(mode="promise_in_bounds"))
        ansvec = jnp.where(iota == r, cand, ansvec)

    res[...] = ansvec
    pltpu.sync_copy(res, out_hbm.at[wid])


def _tc_body(x_ref, o_ref):
    xb = x_ref[...]
    m = jnp.max(xb, axis=1, keepdims=True)
    iota = lax.broadcasted_iota(jnp.int32, (BR, COLS), 1)
    idx = jnp.where(xb == m, iota, COLS)
    o_ref[0, 0, :] = jnp.min(idx, axis=1)


def _argmax_tc(x):
    nb = TC_ROWS // BR
    out = pl.pallas_call(
        _tc_body,
        grid=(nb,),
        in_specs=[pl.BlockSpec((BR, COLS), lambda i: (i, 0))],
        out_specs=pl.BlockSpec((1, 1, BR), lambda i: (i, 0, 0)),
        out_shape=jax.ShapeDtypeStruct((nb, 1, BR), jnp.int32),
    )(x)
    return out.reshape(TC_ROWS)


def kernel(x):
    sc_out = _argmax_sc(x)
    tc_out = _argmax_tc(x)
    sc_idx = sc_out[:, :RPW].reshape(SC_ROWS)
    return jnp.concatenate([tc_out, sc_idx])


# speedup vs baseline: 1.0634x; 1.0634x over previous
import functools

import jax
import jax.numpy as jnp
from jax import lax
from jax.experimental import pallas as pl
from jax.experimental.pallas import tpu as pltpu
from jax.experimental.pallas import tpu_sc as plsc

ROWS = 128
COLS = 32768
NC = 1
NS = 16
L = 16
NW = NC * NS
STEPS = COLS // L

SC_ROWS = 16
SC_BASE = ROWS - SC_ROWS
RPW = SC_ROWS // NW

TC_ROWS = ROWS - SC_ROWS
BR = 56
BC = 128

_mesh = plsc.VectorSubcoreMesh(core_axis_name="c", subcore_axis_name="s", num_cores=1)


@functools.partial(
    pl.kernel,
    mesh=_mesh,
    out_type=jax.ShapeDtypeStruct((NW, L), jnp.int32),
    scratch_types=[
        pltpu.VMEM((2, COLS), jnp.float32),
        pltpu.VMEM((L,), jnp.int32),
        pltpu.SemaphoreType.DMA,
        pltpu.SemaphoreType.DMA,
    ],
)
def _argmax_sc(x_hbm, out_hbm, buf, res, sem0, sem1):
    wid = lax.axis_index("s") * NC + lax.axis_index("c")
    base = SC_BASE + wid * RPW
    sems = (sem0, sem1)

    copies = [pltpu.async_copy(x_hbm.at[base], buf.at[0], sems[0])]
    iota = lax.iota(jnp.int32, L)
    ansvec = jnp.zeros((L,), jnp.int32)

    for r in range(RPW):
        if r + 1 < RPW:
            copies.append(
                pltpu.async_copy(
                    x_hbm.at[base + (r + 1)], buf.at[(r + 1) % 2], sems[(r + 1) % 2]
                )
            )
        copies[r].wait()
        row = buf.at[r % 2]

        def body(j, carry):
            vmax, vstep = carry
            v = row[pl.ds(j * L, L)]
            m = v > vmax
            vmax = jnp.where(m, v, vmax)
            vstep = jnp.where(m, j, vstep)
            return vmax, vstep

        init = (
            jnp.full((L,), -jnp.inf, jnp.float32),
            jnp.zeros((L,), jnp.int32),
        )
        vmax, vstep = lax.fori_loop(0, STEPS, body, init, unroll=8)
        vidx = vstep * L + iota

        gmax = vmax
        for shift in (1, 2, 4, 8):
            perm = iota ^ shift
            gmax = jnp.maximum(gmax, gmax.at[perm].get(mode="promise_in_bounds"))
        cand = jnp.where(vmax == gmax, vidx, COLS)
        for shift in (1, 2, 4, 8):
            perm = iota ^ shift
            cand = jnp.minimum(cand, cand.at[perm].get(mode="promise_in_bounds"))
        ansvec = jnp.where(iota == r, cand, ansvec)

    res[...] = ansvec
    pltpu.sync_copy(res, out_hbm.at[wid])


def _tc_body(x_ref, o_ref):
    xb = x_ref[...]
    m = jnp.max(xb, axis=1, keepdims=True)
    iota = lax.broadcasted_iota(jnp.int32, (BR, COLS), 1)
    idx = jnp.where(xb == m, iota, COLS)
    o_ref[0, 0, :] = jnp.min(idx, axis=1)


def _argmax_tc(x):
    nb = TC_ROWS // BR
    out = pl.pallas_call(
        _tc_body,
        grid=(nb,),
        in_specs=[pl.BlockSpec((BR, COLS), lambda i: (i, 0))],
        out_specs=pl.BlockSpec((1, 1, BR), lambda i: (i, 0, 0)),
        out_shape=jax.ShapeDtypeStruct((nb, 1, BR), jnp.int32),
    )(x)
    return out.reshape(TC_ROWS)


def kernel(x):
    sc_out = _argmax_sc(x)
    tc_out = _argmax_tc(x)
    sc_idx = sc_out[:, :RPW].reshape(SC_ROWS)
    return jnp.concatenate([tc_out, sc_idx])
